# baseline (speedup 1.0000x reference)
import functools

import jax
import jax.numpy as jnp
from jax import lax
from jax.experimental import pallas as pl
from jax.experimental.pallas import tpu as pltpu
from jax.experimental.pallas import tpu_sc as plsc

_B = 4
_C = 96
_HW = 224 * 224
_NC = 2
_NS = 16
_NW = _NC * _NS
_WPB = _NW // _B
_PPW = _HW // _WPB
_CHUNK = 128
_NCHUNK = _PPW // _CHUNK
_GROUPS = _CHUNK // 16

_LN2 = 0.6931471805599453


def _vlog(x):
    bits = lax.bitcast_convert_type(x, jnp.int32)
    e = lax.shift_right_arithmetic(bits, 23) - 127
    m_bits = lax.bitwise_or(lax.bitwise_and(bits, 0x007FFFFF), 0x3F800000)
    m = lax.bitcast_convert_type(m_bits, jnp.float32)
    r = (m - 1.0) / (m + 1.0)
    r2 = r * r
    poly = 1.0 + r2 * (1.0 / 3.0 + r2 * (0.2 + r2 * (1.0 / 7.0 + r2 * (1.0 / 9.0))))
    return e.astype(jnp.float32) * _LN2 + 2.0 * r * poly


def _sc_body(in_hbm, t_hbm, lab_hbm, sums_hbm, cnts_hbm,
             in_buf, t_buf, lab_buf, s_stage, c_stage):
    wid = lax.axis_index("s") * _NC + lax.axis_index("c")
    b = wid // _WPB
    q0 = (wid % _WPB) * _PPW

    zero = jnp.zeros((16,), jnp.float32)

    def chunk_body(k, carry):
        loss_acc, cnt_acc = carry
        q = q0 + k * _CHUNK
        pltpu.sync_copy(in_hbm.at[b, :, pl.ds(q, _CHUNK)], in_buf)
        pltpu.sync_copy(t_hbm.at[b, :, pl.ds(q, _CHUNK)], t_buf)
        pltpu.sync_copy(lab_hbm.at[b, pl.ds(q, _CHUNK)], lab_buf)

        def group_body(g, carry2):
            loss2, cnt2 = carry2
            off = g * 16

            def chan_body(c, carry3):
                si, st, acc = carry3
                iv = in_buf[c, pl.ds(off, 16)]
                tv = t_buf[c, pl.ds(off, 16)]
                te = jnp.exp(tv)
                return (si + jnp.exp(iv), st + te, acc + te * (tv - iv))

            si, st, acc = lax.fori_loop(0, _C, chan_body, (zero, zero, zero))
            kl = acc / st + _vlog(si) - _vlog(st)
            lab = lab_buf[pl.ds(off, 16)]
            mf = jnp.where(lab != 0, 1.0, 0.0).astype(jnp.float32)
            return (loss2 + kl * mf, cnt2 + mf)

        return lax.fori_loop(0, _GROUPS, group_body, (loss_acc, cnt_acc))

    loss_acc, cnt_acc = lax.fori_loop(0, _NCHUNK, chunk_body, (zero, zero))
    s_stage[...] = loss_acc
    c_stage[...] = cnt_acc
    pltpu.sync_copy(s_stage, sums_hbm.at[wid])
    pltpu.sync_copy(c_stage, cnts_hbm.at[wid])


def _finish_body(s_ref, c_ref, o_ref):
    o_ref[...] = (jnp.sum(s_ref[...]) / jnp.sum(c_ref[...]))[None, None]


def kernel(input, target, label):
    in3 = input.reshape(_B, _C, _HW)
    t3 = target.reshape(_B, _C, _HW)
    lab2 = label.reshape(_B, _HW).astype(jnp.int32)

    mesh = plsc.VectorSubcoreMesh(core_axis_name="c", subcore_axis_name="s")
    sc = functools.partial(
        pl.kernel,
        mesh=mesh,
        out_type=[
            jax.ShapeDtypeStruct((_NW, 16), jnp.float32),
            jax.ShapeDtypeStruct((_NW, 16), jnp.float32),
        ],
        scratch_types=[
            pltpu.VMEM((_C, _CHUNK), jnp.float32),
            pltpu.VMEM((_C, _CHUNK), jnp.float32),
            pltpu.VMEM((_CHUNK,), jnp.int32),
            pltpu.VMEM((16,), jnp.float32),
            pltpu.VMEM((16,), jnp.float32),
        ],
    )(_sc_body)
    sums, cnts = sc(in3, t3, lab2)

    loss2d = pl.pallas_call(
        _finish_body,
        out_shape=jax.ShapeDtypeStruct((1, 1), jnp.float32),
    )(sums, cnts)
    return loss2d[0, 0]


# speedup vs baseline: 1.2384x; 1.2384x over previous
import functools

import jax
import jax.numpy as jnp
from jax import lax
from jax.experimental import pallas as pl
from jax.experimental.pallas import tpu as pltpu
from jax.experimental.pallas import tpu_sc as plsc

_B = 4
_C = 96
_HW = 224 * 224
_NC = 2
_NS = 16
_NW = _NC * _NS
_WPB = _NW // _B
_PPW = _HW // _WPB
_CHUNK = 128
_NCHUNK = _PPW // _CHUNK
_GROUPS = _CHUNK // 16

_LN2 = 0.6931471805599453


def _vlog(x):
    bits = lax.bitcast_convert_type(x, jnp.int32)
    e = lax.shift_right_arithmetic(bits, 23) - 127
    m_bits = lax.bitwise_or(lax.bitwise_and(bits, 0x007FFFFF), 0x3F800000)
    m = lax.bitcast_convert_type(m_bits, jnp.float32)
    r = (m - 1.0) / (m + 1.0)
    r2 = r * r
    poly = 1.0 + r2 * (1.0 / 3.0 + r2 * (0.2 + r2 * (1.0 / 7.0 + r2 * (1.0 / 9.0))))
    return e.astype(jnp.float32) * _LN2 + 2.0 * r * poly


def _sc_body(in_hbm, t_hbm, lab_hbm, sums_hbm, cnts_hbm,
             in_buf, t_buf, lab_buf, s_stage, c_stage):
    wid = lax.axis_index("s") * _NC + lax.axis_index("c")
    b = wid // _WPB
    q0 = (wid % _WPB) * _PPW

    zero = jnp.zeros((16,), jnp.float32)

    def chunk_body(k, carry):
        loss_acc, cnt_acc = carry
        q = q0 + k * _CHUNK
        pltpu.sync_copy(in_hbm.at[b, :, pl.ds(q, _CHUNK)], in_buf)
        pltpu.sync_copy(t_hbm.at[b, :, pl.ds(q, _CHUNK)], t_buf)
        pltpu.sync_copy(lab_hbm.at[b, pl.ds(q, _CHUNK)], lab_buf)

        for g in range(_GROUPS):
            off = g * 16

            def chan_body(c, carry3):
                si, st, acc = carry3
                for u in range(4):
                    cc = c * 4 + u
                    iv = in_buf[cc, pl.ds(off, 16)]
                    tv = t_buf[cc, pl.ds(off, 16)]
                    te = jnp.exp(tv)
                    si = si + jnp.exp(iv)
                    st = st + te
                    acc = acc + te * (tv - iv)
                return (si, st, acc)

            si, st, acc = lax.fori_loop(0, _C // 4, chan_body,
                                        (zero, zero, zero))
            kl = acc / st + _vlog(si) - _vlog(st)
            lab = lab_buf[pl.ds(off, 16)]
            mf = jnp.where(lab != 0, 1.0, 0.0).astype(jnp.float32)
            loss_acc = loss_acc + kl * mf
            cnt_acc = cnt_acc + mf

        return (loss_acc, cnt_acc)

    loss_acc, cnt_acc = lax.fori_loop(0, _NCHUNK, chunk_body, (zero, zero))
    s_stage[...] = loss_acc
    c_stage[...] = cnt_acc
    pltpu.sync_copy(s_stage, sums_hbm.at[wid])
    pltpu.sync_copy(c_stage, cnts_hbm.at[wid])


def _finish_body(s_ref, c_ref, o_ref):
    o_ref[...] = (jnp.sum(s_ref[...]) / jnp.sum(c_ref[...]))[None, None]


def kernel(input, target, label):
    in3 = input.reshape(_B, _C, _HW)
    t3 = target.reshape(_B, _C, _HW)
    lab2 = label.reshape(_B, _HW).astype(jnp.int32)

    mesh = plsc.VectorSubcoreMesh(core_axis_name="c", subcore_axis_name="s")
    sc = functools.partial(
        pl.kernel,
        mesh=mesh,
        out_type=[
            jax.ShapeDtypeStruct((_NW, 16), jnp.float32),
            jax.ShapeDtypeStruct((_NW, 16), jnp.float32),
        ],
        scratch_types=[
            pltpu.VMEM((_C, _CHUNK), jnp.float32),
            pltpu.VMEM((_C, _CHUNK), jnp.float32),
            pltpu.VMEM((_CHUNK,), jnp.int32),
            pltpu.VMEM((16,), jnp.float32),
            pltpu.VMEM((16,), jnp.float32),
        ],
    )(_sc_body)
    sums, cnts = sc(in3, t3, lab2)

    loss2d = pl.pallas_call(
        _finish_body,
        out_shape=jax.ShapeDtypeStruct((1, 1), jnp.float32),
    )(sums, cnts)
    return loss2d[0, 0]


# speedup vs baseline: 1.7630x; 1.4236x over previous
import functools

import jax
import jax.numpy as jnp
from jax import lax
from jax.experimental import pallas as pl
from jax.experimental.pallas import tpu as pltpu
from jax.experimental.pallas import tpu_sc as plsc

_B = 4
_C = 96
_HW = 224 * 224
_NC = 2
_NS = 16
_NW = _NC * _NS
_WPB = _NW // _B
_PPW = _HW // _WPB
_CHUNK = 128
_NCHUNK = _PPW // _CHUNK
_GROUPS = _CHUNK // 16

_LN2 = 0.6931471805599453


def _vlog(x):
    bits = lax.bitcast_convert_type(x, jnp.int32)
    e = lax.shift_right_arithmetic(bits, 23) - 127
    m_bits = lax.bitwise_or(lax.bitwise_and(bits, 0x007FFFFF), 0x3F800000)
    m = lax.bitcast_convert_type(m_bits, jnp.float32)
    r = (m - 1.0) / (m + 1.0)
    r2 = r * r
    poly = 1.0 + r2 * (1.0 / 3.0 + r2 * (0.2 + r2 * (1.0 / 7.0 + r2 * (1.0 / 9.0))))
    return e.astype(jnp.float32) * _LN2 + 2.0 * r * poly


def _sc_body(in_hbm, t_hbm, lab_hbm, sums_hbm, cnts_hbm,
             in0, t0, lab0, in1, t1, lab1, s_stage, c_stage, sem0, sem1):
    wid = lax.axis_index("s") * _NC + lax.axis_index("c")
    b = wid // _WPB
    q0 = (wid % _WPB) * _PPW

    zero = jnp.zeros((16,), jnp.float32)

    def issue(q, ib, tb, lb, sem):
        pltpu.async_copy(in_hbm.at[b, :, pl.ds(q, _CHUNK)], ib, sem)
        pltpu.async_copy(t_hbm.at[b, :, pl.ds(q, _CHUNK)], tb, sem)
        pltpu.async_copy(lab_hbm.at[b, pl.ds(q, _CHUNK)], lb, sem)

    def drain(q, ib, tb, lb, sem):
        pltpu.make_async_copy(in_hbm.at[b, :, pl.ds(q, _CHUNK)], ib, sem).wait()
        pltpu.make_async_copy(t_hbm.at[b, :, pl.ds(q, _CHUNK)], tb, sem).wait()
        pltpu.make_async_copy(lab_hbm.at[b, pl.ds(q, _CHUNK)], lb, sem).wait()

    def compute(in_buf, t_buf, lab_buf, loss_acc, cnt_acc):
        for g in range(_GROUPS):
            off = g * 16

            def chan_body(c, carry3):
                si, st, acc = carry3
                for u in range(4):
                    cc = c * 4 + u
                    iv = in_buf[cc, pl.ds(off, 16)]
                    tv = t_buf[cc, pl.ds(off, 16)]
                    te = jnp.exp(tv)
                    si = si + jnp.exp(iv)
                    st = st + te
                    acc = acc + te * (tv - iv)
                return (si, st, acc)

            si, st, acc = lax.fori_loop(0, _C // 4, chan_body,
                                        (zero, zero, zero))
            kl = acc / st + _vlog(si) - _vlog(st)
            lab = lab_buf[pl.ds(off, 16)]
            mf = jnp.where(lab != 0, 1.0, 0.0).astype(jnp.float32)
            loss_acc = loss_acc + kl * mf
            cnt_acc = cnt_acc + mf
        return loss_acc, cnt_acc

    issue(q0, in0, t0, lab0, sem0)

    def pair_body(k, carry):
        loss, cnt = carry
        qA = q0 + (2 * k) * _CHUNK
        qB = qA + _CHUNK
        qN = qB + _CHUNK
        drain(qA, in0, t0, lab0, sem0)
        issue(qB, in1, t1, lab1, sem1)
        loss, cnt = compute(in0, t0, lab0, loss, cnt)
        drain(qB, in1, t1, lab1, sem1)
        issue(qN, in0, t0, lab0, sem0)
        loss, cnt = compute(in1, t1, lab1, loss, cnt)
        return loss, cnt

    loss_acc, cnt_acc = lax.fori_loop(0, (_NCHUNK - 1) // 2, pair_body,
                                      (zero, zero))
    qL = q0 + (_NCHUNK - 1) * _CHUNK
    drain(qL, in0, t0, lab0, sem0)
    loss_acc, cnt_acc = compute(in0, t0, lab0, loss_acc, cnt_acc)
    s_stage[...] = loss_acc
    c_stage[...] = cnt_acc
    pltpu.sync_copy(s_stage, sums_hbm.at[wid])
    pltpu.sync_copy(c_stage, cnts_hbm.at[wid])


def _finish_body(s_ref, c_ref, o_ref):
    o_ref[...] = (jnp.sum(s_ref[...]) / jnp.sum(c_ref[...]))[None, None]


def kernel(input, target, label):
    in3 = input.reshape(_B, _C, _HW)
    t3 = target.reshape(_B, _C, _HW)
    lab2 = label.reshape(_B, _HW).astype(jnp.int32)

    mesh = plsc.VectorSubcoreMesh(core_axis_name="c", subcore_axis_name="s")
    sc = functools.partial(
        pl.kernel,
        mesh=mesh,
        out_type=[
            jax.ShapeDtypeStruct((_NW, 16), jnp.float32),
            jax.ShapeDtypeStruct((_NW, 16), jnp.float32),
        ],
        scratch_types=[
            pltpu.VMEM((_C, _CHUNK), jnp.float32),
            pltpu.VMEM((_C, _CHUNK), jnp.float32),
            pltpu.VMEM((_CHUNK,), jnp.int32),
            pltpu.VMEM((_C, _CHUNK), jnp.float32),
            pltpu.VMEM((_C, _CHUNK), jnp.float32),
            pltpu.VMEM((_CHUNK,), jnp.int32),
            pltpu.VMEM((16,), jnp.float32),
            pltpu.VMEM((16,), jnp.float32),
            pltpu.SemaphoreType.DMA,
            pltpu.SemaphoreType.DMA,
        ],
    )(_sc_body)
    sums, cnts = sc(in3, t3, lab2)

    loss2d = pl.pallas_call(
        _finish_body,
        out_shape=jax.ShapeDtypeStruct((1, 1), jnp.float32),
    )(sums, cnts)
    return loss2d[0, 0]


# speedup vs baseline: 1.7746x; 1.0066x over previous
import functools

import jax
import jax.numpy as jnp
from jax import lax
from jax.experimental import pallas as pl
from jax.experimental.pallas import tpu as pltpu
from jax.experimental.pallas import tpu_sc as plsc

_B = 4
_C = 96
_HW = 224 * 224
_NC = 2
_NS = 16
_NW = _NC * _NS
_WPB = _NW // _B
_PPW = _HW // _WPB
_CHUNK = 128
_NCHUNK = _PPW // _CHUNK
_GROUPS = _CHUNK // 16

_LN2 = 0.6931471805599453


def _vlog(x):
    bits = lax.bitcast_convert_type(x, jnp.int32)
    e = lax.shift_right_arithmetic(bits, 23) - 127
    m_bits = lax.bitwise_or(lax.bitwise_and(bits, 0x007FFFFF), 0x3F800000)
    m = lax.bitcast_convert_type(m_bits, jnp.float32)
    r = (m - 1.0) / (m + 1.0)
    r2 = r * r
    poly = 1.0 + r2 * (1.0 / 3.0 + r2 * (0.2 + r2 * (1.0 / 7.0 + r2 * (1.0 / 9.0))))
    return e.astype(jnp.float32) * _LN2 + 2.0 * r * poly


def _sc_body(in_hbm, t_hbm, lab_hbm, sums_hbm, cnts_hbm,
             in0, t0, lab0, in1, t1, lab1, idx_buf, s_stage, c_stage,
             sem0, sem1):
    wid = lax.axis_index("s") * _NC + lax.axis_index("c")
    b = wid // _WPB
    q0 = (wid % _WPB) * _PPW

    zero = jnp.zeros((16,), jnp.float32)
    lane = lax.iota(jnp.int32, 16)
    for g in range(_GROUPS):
        idx_buf[pl.ds(g * 16, 16)] = jnp.zeros((16,), jnp.int32)

    def issue(q, ib, tb, lb, sem):
        pltpu.async_copy(in_hbm.at[b, :, pl.ds(q, _CHUNK)], ib, sem)
        pltpu.async_copy(t_hbm.at[b, :, pl.ds(q, _CHUNK)], tb, sem)
        pltpu.async_copy(lab_hbm.at[b, pl.ds(q, _CHUNK)], lb, sem)

    def drain(q, ib, tb, lb, sem):
        pltpu.make_async_copy(in_hbm.at[b, :, pl.ds(q, _CHUNK)], ib, sem).wait()
        pltpu.make_async_copy(t_hbm.at[b, :, pl.ds(q, _CHUNK)], tb, sem).wait()
        pltpu.make_async_copy(lab_hbm.at[b, pl.ds(q, _CHUNK)], lb, sem).wait()

    def compute(in_buf, t_buf, lab_buf, loss_acc, cnt_acc):
        total = jnp.int32(0)
        for g in range(_GROUPS):
            lv = lab_buf[pl.ds(g * 16, 16)]
            m = lv != 0
            mi = jnp.where(m, 1, 0).astype(jnp.int32)
            pos = plsc.cumsum(mi) - 1 + total
            plsc.store_scatter(idx_buf, [pos], lane + (g * 16), mask=m)
            total = total + jnp.sum(mi)
        ng = lax.shift_right_arithmetic(total + 15, 4)

        def gbody(g, carry2):
            loss2, cnt2 = carry2
            base = g * 16
            pidx = idx_buf[pl.ds(base, 16)]

            zero_i = jnp.zeros((16,), jnp.int32)

            @plsc.parallel_loop(0, _C, step=1, unroll=8,
                                carry=(zero, zero, zero, zero_i))
            def chan_body(c, carry3):
                si, st, acc, cv = carry3
                iv = plsc.load_gather(in_buf, [cv, pidx])
                tv = plsc.load_gather(t_buf, [cv, pidx])
                te = jnp.exp(tv)
                return (si + jnp.exp(iv), st + te, acc + te * (tv - iv),
                        cv + 1)

            si, st, acc, _ = chan_body
            kl = acc / st + _vlog(si) - _vlog(st)
            mf = jnp.where(lane + base < total, 1.0, 0.0)
            return (loss2 + kl * mf, cnt2 + mf)

        return lax.fori_loop(0, ng, gbody, (loss_acc, cnt_acc))

    issue(q0, in0, t0, lab0, sem0)

    def pair_body(k, carry):
        loss, cnt = carry
        qA = q0 + (2 * k) * _CHUNK
        qB = qA + _CHUNK
        qN = qB + _CHUNK
        drain(qA, in0, t0, lab0, sem0)
        issue(qB, in1, t1, lab1, sem1)
        loss, cnt = compute(in0, t0, lab0, loss, cnt)
        drain(qB, in1, t1, lab1, sem1)
        issue(qN, in0, t0, lab0, sem0)
        loss, cnt = compute(in1, t1, lab1, loss, cnt)
        return loss, cnt

    loss_acc, cnt_acc = lax.fori_loop(0, (_NCHUNK - 1) // 2, pair_body,
                                      (zero, zero))
    qL = q0 + (_NCHUNK - 1) * _CHUNK
    drain(qL, in0, t0, lab0, sem0)
    loss_acc, cnt_acc = compute(in0, t0, lab0, loss_acc, cnt_acc)
    s_stage[...] = loss_acc
    c_stage[...] = cnt_acc
    pltpu.sync_copy(s_stage, sums_hbm.at[wid])
    pltpu.sync_copy(c_stage, cnts_hbm.at[wid])


def _finish_body(s_ref, c_ref, o_ref):
    o_ref[...] = (jnp.sum(s_ref[...]) / jnp.sum(c_ref[...]))[None, None]


def kernel(input, target, label):
    in3 = input.reshape(_B, _C, _HW)
    t3 = target.reshape(_B, _C, _HW)
    lab2 = label.reshape(_B, _HW).astype(jnp.int32)

    mesh = plsc.VectorSubcoreMesh(core_axis_name="c", subcore_axis_name="s")
    sc = functools.partial(
        pl.kernel,
        mesh=mesh,
        compiler_params=pltpu.CompilerParams(needs_layout_passes=False),
        out_type=[
            jax.ShapeDtypeStruct((_NW, 16), jnp.float32),
            jax.ShapeDtypeStruct((_NW, 16), jnp.float32),
        ],
        scratch_types=[
            pltpu.VMEM((_C, _CHUNK), jnp.float32),
            pltpu.VMEM((_C, _CHUNK), jnp.float32),
            pltpu.VMEM((_CHUNK,), jnp.int32),
            pltpu.VMEM((_C, _CHUNK), jnp.float32),
            pltpu.VMEM((_C, _CHUNK), jnp.float32),
            pltpu.VMEM((_CHUNK,), jnp.int32),
            pltpu.VMEM((_CHUNK,), jnp.int32),
            pltpu.VMEM((16,), jnp.float32),
            pltpu.VMEM((16,), jnp.float32),
            pltpu.SemaphoreType.DMA,
            pltpu.SemaphoreType.DMA,
        ],
    )(_sc_body)
    sums, cnts = sc(in3, t3, lab2)

    loss2d = pl.pallas_call(
        _finish_body,
        out_shape=jax.ShapeDtypeStruct((1, 1), jnp.float32),
    )(sums, cnts)
    return loss2d[0, 0]


# speedup vs baseline: 4.3303x; 2.4402x over previous
import functools

import jax
import jax.numpy as jnp
from jax import lax
from jax.experimental import pallas as pl
from jax.experimental.pallas import tpu as pltpu
from jax.experimental.pallas import tpu_sc as plsc

_B = 4
_C = 96
_H = 224
_W = 224
_NE = 8
_CE = _C // _NE
_RB = 8
_NRB = _H // _RB
_NG = _W // 16


def _p1_body(in_hbm, t_hbm, si_hbm, st_hbm, ac_hbm,
             i0, t0, i1, t1, a_si, a_st, a_ac, b_si, b_st, b_ac,
             semi0, semi1, semo0, semo1):
    wid = lax.axis_index("s") * 2 + lax.axis_index("c")
    b = wid // _NE
    e = wid % _NE
    c0 = e * _CE

    def issue_in(rb, ib, tb, sem):
        pltpu.async_copy(
            in_hbm.at[b, pl.ds(c0, _CE), pl.ds(rb * _RB, _RB), :], ib, sem)
        pltpu.async_copy(
            t_hbm.at[b, pl.ds(c0, _CE), pl.ds(rb * _RB, _RB), :], tb, sem)

    def drain_in(rb, ib, tb, sem):
        pltpu.make_async_copy(
            in_hbm.at[b, pl.ds(c0, _CE), pl.ds(rb * _RB, _RB), :], ib,
            sem).wait()
        pltpu.make_async_copy(
            t_hbm.at[b, pl.ds(c0, _CE), pl.ds(rb * _RB, _RB), :], tb,
            sem).wait()

    def issue_out(rb, s_si, s_st, s_ac, sem):
        pltpu.async_copy(s_si, si_hbm.at[e, b, pl.ds(rb * _RB, _RB), :], sem)
        pltpu.async_copy(s_st, st_hbm.at[e, b, pl.ds(rb * _RB, _RB), :], sem)
        pltpu.async_copy(s_ac, ac_hbm.at[e, b, pl.ds(rb * _RB, _RB), :], sem)

    def drain_out(s_si, s_st, s_ac, sem):
        pltpu.make_async_copy(s_si, si_hbm.at[e, b, pl.ds(0, _RB), :],
                              sem).wait()
        pltpu.make_async_copy(s_st, st_hbm.at[e, b, pl.ds(0, _RB), :],
                              sem).wait()
        pltpu.make_async_copy(s_ac, ac_hbm.at[e, b, pl.ds(0, _RB), :],
                              sem).wait()

    zero = jnp.zeros((16,), jnp.float32)

    def compute(ib, tb, s_si, s_st, s_ac):
        def row_body(r, carry):
            @plsc.parallel_loop(0, _NG, step=1, unroll=2,
                                carry=jnp.int32(0))
            def grp_body(o, dummy):
                off = o * 16
                si = zero
                st = zero
                ac = zero
                for c in range(_CE):
                    iv = ib[c, r, pl.ds(off, 16)]
                    tv = tb[c, r, pl.ds(off, 16)]
                    te = jnp.exp(tv)
                    si = si + jnp.exp(iv)
                    st = st + te
                    ac = ac + te * (tv - iv)
                s_si[r, pl.ds(off, 16)] = si
                s_st[r, pl.ds(off, 16)] = st
                s_ac[r, pl.ds(off, 16)] = ac
                return dummy

            return carry

        lax.fori_loop(0, _RB, row_body, jnp.int32(0))

    issue_in(0, i0, t0, semi0)
    drain_in(0, i0, t0, semi0)
    issue_in(1, i1, t1, semi1)
    compute(i0, t0, a_si, a_st, a_ac)
    issue_out(0, a_si, a_st, a_ac, semo0)
    drain_in(1, i1, t1, semi1)
    issue_in(2, i0, t0, semi0)
    compute(i1, t1, b_si, b_st, b_ac)
    issue_out(1, b_si, b_st, b_ac, semo1)

    def pair_body(k, carry):
        rb_a = 2 * k
        rb_b = rb_a + 1
        rb_n = jnp.minimum(rb_a + 2, _NRB - 1)
        drain_in(rb_a, i0, t0, semi0)
        issue_in(rb_b, i1, t1, semi1)
        drain_out(a_si, a_st, a_ac, semo0)
        compute(i0, t0, a_si, a_st, a_ac)
        issue_out(rb_a, a_si, a_st, a_ac, semo0)
        drain_in(rb_b, i1, t1, semi1)
        issue_in(rb_n, i0, t0, semi0)
        drain_out(b_si, b_st, b_ac, semo1)
        compute(i1, t1, b_si, b_st, b_ac)
        issue_out(rb_b, b_si, b_st, b_ac, semo1)
        return carry

    lax.fori_loop(1, _NRB // 2, pair_body, jnp.int32(0))
    drain_in(_NRB - 1, i0, t0, semi0)
    drain_out(a_si, a_st, a_ac, semo0)
    drain_out(b_si, b_st, b_ac, semo1)


def _p2_body(si_ref, st_ref, ac_ref, lab_ref, s_ref, n_ref, l_ref):
    i = pl.program_id(0)
    j = pl.program_id(1)

    @pl.when((i == 0) & (j == 0))
    def _():
        s_ref[...] = jnp.zeros_like(s_ref)
        n_ref[...] = jnp.zeros_like(n_ref)

    si = jnp.sum(si_ref[...], axis=0)
    st = jnp.sum(st_ref[...], axis=0)
    ac = jnp.sum(ac_ref[...], axis=0)
    kl = ac / st + jnp.log(si) - jnp.log(st)
    m = lab_ref[...] != 0
    s_ref[...] += jnp.sum(jnp.where(m, kl, 0.0))[None, None]
    n_ref[...] += jnp.sum(jnp.where(m, 1.0, 0.0))[None, None]

    @pl.when((i == _B - 1) & (j == pl.num_programs(1) - 1))
    def _():
        l_ref[...] = s_ref[...] / n_ref[...]


def kernel(input, target, label):
    lab = label.astype(jnp.int32)

    mesh = plsc.VectorSubcoreMesh(core_axis_name="c", subcore_axis_name="s")
    p1 = functools.partial(
        pl.kernel,
        mesh=mesh,
        out_type=[
            jax.ShapeDtypeStruct((_NE, _B, _H, _W), jnp.float32),
            jax.ShapeDtypeStruct((_NE, _B, _H, _W), jnp.float32),
            jax.ShapeDtypeStruct((_NE, _B, _H, _W), jnp.float32),
        ],
        scratch_types=[
            pltpu.VMEM((_CE, _RB, _W), jnp.float32),
            pltpu.VMEM((_CE, _RB, _W), jnp.float32),
            pltpu.VMEM((_CE, _RB, _W), jnp.float32),
            pltpu.VMEM((_CE, _RB, _W), jnp.float32),
            pltpu.VMEM((_RB, _W), jnp.float32),
            pltpu.VMEM((_RB, _W), jnp.float32),
            pltpu.VMEM((_RB, _W), jnp.float32),
            pltpu.VMEM((_RB, _W), jnp.float32),
            pltpu.VMEM((_RB, _W), jnp.float32),
            pltpu.VMEM((_RB, _W), jnp.float32),
            pltpu.SemaphoreType.DMA,
            pltpu.SemaphoreType.DMA,
            pltpu.SemaphoreType.DMA,
            pltpu.SemaphoreType.DMA,
        ],
    )(_p1_body)
    si_p, st_p, ac_p = p1(input, target)

    rows = 56
    grid = (_B, _H // rows)
    _, _, loss2d = pl.pallas_call(
        _p2_body,
        grid=grid,
        in_specs=[
            pl.BlockSpec((_NE, 1, rows, _W), lambda i, j: (0, i, j, 0)),
            pl.BlockSpec((_NE, 1, rows, _W), lambda i, j: (0, i, j, 0)),
            pl.BlockSpec((_NE, 1, rows, _W), lambda i, j: (0, i, j, 0)),
            pl.BlockSpec((1, rows, _W), lambda i, j: (i, j, 0)),
        ],
        out_specs=[
            pl.BlockSpec((1, 1), lambda i, j: (0, 0)),
            pl.BlockSpec((1, 1), lambda i, j: (0, 0)),
            pl.BlockSpec((1, 1), lambda i, j: (0, 0)),
        ],
        out_shape=[
            jax.ShapeDtypeStruct((1, 1), jnp.float32),
            jax.ShapeDtypeStruct((1, 1), jnp.float32),
            jax.ShapeDtypeStruct((1, 1), jnp.float32),
        ],
    )(si_p, st_p, ac_p, lab)
    return loss2d[0, 0]


# speedup vs baseline: 5.2401x; 1.2101x over previous
import functools

import jax
import jax.numpy as jnp
from jax import lax
from jax.experimental import pallas as pl
from jax.experimental.pallas import tpu as pltpu
from jax.experimental.pallas import tpu_sc as plsc

_B = 4
_C = 96
_H = 224
_W = 224
_NE = 8
_CE = _C // _NE
_RB = 8
_RS = 128
_NRB = _RS // _RB
_NG = _W // 16
_TCH = 32


def _p1_body(in_hbm, t_hbm, si_hbm, st_hbm, ac_hbm,
             i0, t0, i1, t1, a_si, a_st, a_ac, b_si, b_st, b_ac,
             semi0, semi1, semo0, semo1):
    wid = lax.axis_index("s") * 2 + lax.axis_index("c")
    b = wid // _NE
    e = wid % _NE
    c0 = e * _CE

    def issue_in(rb, ib, tb, sem):
        pltpu.async_copy(
            in_hbm.at[b, pl.ds(c0, _CE), pl.ds(rb * _RB, _RB), :], ib, sem)
        pltpu.async_copy(
            t_hbm.at[b, pl.ds(c0, _CE), pl.ds(rb * _RB, _RB), :], tb, sem)

    def drain_in(rb, ib, tb, sem):
        pltpu.make_async_copy(
            in_hbm.at[b, pl.ds(c0, _CE), pl.ds(rb * _RB, _RB), :], ib,
            sem).wait()
        pltpu.make_async_copy(
            t_hbm.at[b, pl.ds(c0, _CE), pl.ds(rb * _RB, _RB), :], tb,
            sem).wait()

    def issue_out(rb, s_si, s_st, s_ac, sem):
        pltpu.async_copy(s_si, si_hbm.at[e, b, pl.ds(rb * _RB, _RB), :], sem)
        pltpu.async_copy(s_st, st_hbm.at[e, b, pl.ds(rb * _RB, _RB), :], sem)
        pltpu.async_copy(s_ac, ac_hbm.at[e, b, pl.ds(rb * _RB, _RB), :], sem)

    def drain_out(s_si, s_st, s_ac, sem):
        pltpu.make_async_copy(s_si, si_hbm.at[e, b, pl.ds(0, _RB), :],
                              sem).wait()
        pltpu.make_async_copy(s_st, st_hbm.at[e, b, pl.ds(0, _RB), :],
                              sem).wait()
        pltpu.make_async_copy(s_ac, ac_hbm.at[e, b, pl.ds(0, _RB), :],
                              sem).wait()

    zero = jnp.zeros((16,), jnp.float32)

    def compute(ib, tb, s_si, s_st, s_ac):
        def row_body(r, carry):
            @plsc.parallel_loop(0, _NG, step=1, unroll=2,
                                carry=jnp.int32(0))
            def grp_body(o, dummy):
                off = o * 16
                si = zero
                st = zero
                ac = zero
                for c in range(_CE):
                    iv = ib[c, r, pl.ds(off, 16)]
                    tv = tb[c, r, pl.ds(off, 16)]
                    te = jnp.exp(tv)
                    si = si + jnp.exp(iv)
                    st = st + te
                    ac = ac + te * (tv - iv)
                s_si[r, pl.ds(off, 16)] = si
                s_st[r, pl.ds(off, 16)] = st
                s_ac[r, pl.ds(off, 16)] = ac
                return dummy

            return carry

        lax.fori_loop(0, _RB, row_body, jnp.int32(0))

    issue_in(0, i0, t0, semi0)
    drain_in(0, i0, t0, semi0)
    issue_in(1, i1, t1, semi1)
    compute(i0, t0, a_si, a_st, a_ac)
    issue_out(0, a_si, a_st, a_ac, semo0)
    drain_in(1, i1, t1, semi1)
    issue_in(2, i0, t0, semi0)
    compute(i1, t1, b_si, b_st, b_ac)
    issue_out(1, b_si, b_st, b_ac, semo1)

    def pair_body(k, carry):
        rb_a = 2 * k
        rb_b = rb_a + 1
        rb_n = jnp.minimum(rb_a + 2, _NRB - 1)
        drain_in(rb_a, i0, t0, semi0)
        issue_in(rb_b, i1, t1, semi1)
        drain_out(a_si, a_st, a_ac, semo0)
        compute(i0, t0, a_si, a_st, a_ac)
        issue_out(rb_a, a_si, a_st, a_ac, semo0)
        drain_in(rb_b, i1, t1, semi1)
        issue_in(rb_n, i0, t0, semi0)
        drain_out(b_si, b_st, b_ac, semo1)
        compute(i1, t1, b_si, b_st, b_ac)
        issue_out(rb_b, b_si, b_st, b_ac, semo1)
        return carry

    lax.fori_loop(1, _NRB // 2, pair_body, jnp.int32(0))
    drain_in(_NRB - 1, i0, t0, semi0)
    drain_out(a_si, a_st, a_ac, semo0)
    drain_out(b_si, b_st, b_ac, semo1)


def _tc_body(in_ref, t_ref, lab_ref, s_ref, n_ref):
    i = pl.program_id(0)
    j = pl.program_id(1)

    @pl.when((i == 0) & (j == 0))
    def _():
        s_ref[...] = jnp.zeros_like(s_ref)
        n_ref[...] = jnp.zeros_like(n_ref)

    iv = in_ref[0]
    tv = t_ref[0]
    te = jnp.exp(tv)
    si = jnp.sum(jnp.exp(iv), axis=0)
    st = jnp.sum(te, axis=0)
    ac = jnp.sum(te * (tv - iv), axis=0)
    kl = ac / st + jnp.log(si) - jnp.log(st)
    m = lab_ref[0] != 0
    s_ref[...] += jnp.sum(jnp.where(m, kl, 0.0))[None, None]
    n_ref[...] += jnp.sum(jnp.where(m, 1.0, 0.0))[None, None]


def _p2_body(si_ref, st_ref, ac_ref, lab_ref, ts_ref, tn_ref,
             s_ref, n_ref, l_ref):
    i = pl.program_id(0)
    j = pl.program_id(1)

    @pl.when((i == 0) & (j == 0))
    def _():
        s_ref[...] = jnp.zeros_like(s_ref)
        n_ref[...] = jnp.zeros_like(n_ref)

    si = jnp.sum(si_ref[...], axis=0)
    st = jnp.sum(st_ref[...], axis=0)
    ac = jnp.sum(ac_ref[...], axis=0)
    kl = ac / st + jnp.log(si) - jnp.log(st)
    m = lab_ref[...] != 0
    s_ref[...] += jnp.sum(jnp.where(m, kl, 0.0))[None, None]
    n_ref[...] += jnp.sum(jnp.where(m, 1.0, 0.0))[None, None]

    @pl.when((i == _B - 1) & (j == pl.num_programs(1) - 1))
    def _():
        l_ref[...] = ((s_ref[...] + ts_ref[...])
                      / (n_ref[...] + tn_ref[...]))


def kernel(input, target, label):
    lab = label.astype(jnp.int32)

    mesh = plsc.VectorSubcoreMesh(core_axis_name="c", subcore_axis_name="s")
    p1 = functools.partial(
        pl.kernel,
        mesh=mesh,
        out_type=[
            jax.ShapeDtypeStruct((_NE, _B, _RS, _W), jnp.float32),
            jax.ShapeDtypeStruct((_NE, _B, _RS, _W), jnp.float32),
            jax.ShapeDtypeStruct((_NE, _B, _RS, _W), jnp.float32),
        ],
        scratch_types=[
            pltpu.VMEM((_CE, _RB, _W), jnp.float32),
            pltpu.VMEM((_CE, _RB, _W), jnp.float32),
            pltpu.VMEM((_CE, _RB, _W), jnp.float32),
            pltpu.VMEM((_CE, _RB, _W), jnp.float32),
            pltpu.VMEM((_RB, _W), jnp.float32),
            pltpu.VMEM((_RB, _W), jnp.float32),
            pltpu.VMEM((_RB, _W), jnp.float32),
            pltpu.VMEM((_RB, _W), jnp.float32),
            pltpu.VMEM((_RB, _W), jnp.float32),
            pltpu.VMEM((_RB, _W), jnp.float32),
            pltpu.SemaphoreType.DMA,
            pltpu.SemaphoreType.DMA,
            pltpu.SemaphoreType.DMA,
            pltpu.SemaphoreType.DMA,
        ],
    )(_p1_body)
    si_p, st_p, ac_p = p1(input, target)

    nb = (_H - _RS) // _TCH
    ts2d, tn2d = pl.pallas_call(
        _tc_body,
        grid=(_B, nb),
        in_specs=[
            pl.BlockSpec((1, _C, _TCH, _W),
                         lambda i, j: (i, 0, _RS // _TCH + j, 0)),
            pl.BlockSpec((1, _C, _TCH, _W),
                         lambda i, j: (i, 0, _RS // _TCH + j, 0)),
            pl.BlockSpec((1, _TCH, _W), lambda i, j: (i, _RS // _TCH + j, 0)),
        ],
        out_specs=[
            pl.BlockSpec((1, 1), lambda i, j: (0, 0)),
            pl.BlockSpec((1, 1), lambda i, j: (0, 0)),
        ],
        out_shape=[
            jax.ShapeDtypeStruct((1, 1), jnp.float32),
            jax.ShapeDtypeStruct((1, 1), jnp.float32),
        ],
    )(input, target, lab)

    rows = 64
    grid = (_B, _RS // rows)
    _, _, loss2d = pl.pallas_call(
        _p2_body,
        grid=grid,
        in_specs=[
            pl.BlockSpec((_NE, 1, rows, _W), lambda i, j: (0, i, j, 0)),
            pl.BlockSpec((_NE, 1, rows, _W), lambda i, j: (0, i, j, 0)),
            pl.BlockSpec((_NE, 1, rows, _W), lambda i, j: (0, i, j, 0)),
            pl.BlockSpec((1, rows, _W), lambda i, j: (i, j, 0)),
            pl.BlockSpec((1, 1), lambda i, j: (0, 0)),
            pl.BlockSpec((1, 1), lambda i, j: (0, 0)),
        ],
        out_specs=[
            pl.BlockSpec((1, 1), lambda i, j: (0, 0)),
            pl.BlockSpec((1, 1), lambda i, j: (0, 0)),
            pl.BlockSpec((1, 1), lambda i, j: (0, 0)),
        ],
        out_shape=[
            jax.ShapeDtypeStruct((1, 1), jnp.float32),
            jax.ShapeDtypeStruct((1, 1), jnp.float32),
            jax.ShapeDtypeStruct((1, 1), jnp.float32),
        ],
    )(si_p, st_p, ac_p, lab, ts2d, tn2d)
    return loss2d[0, 0]


# speedup vs baseline: 5.4977x; 1.0492x over previous
import functools

import jax
import jax.numpy as jnp
from jax import lax
from jax.experimental import pallas as pl
from jax.experimental.pallas import tpu as pltpu
from jax.experimental.pallas import tpu_sc as plsc

_B = 4
_C = 96
_H = 224
_W = 224
_NE = 8
_CE = _C // _NE
_RB = 8
_RS = 112
_NRB = _RS // _RB
_NG = _W // 16
_TCH = 56


def _p1_body(in_hbm, t_hbm, si_hbm, st_hbm, ac_hbm,
             i0, t0, i1, t1, a_si, a_st, a_ac, b_si, b_st, b_ac,
             semi0, semi1, semo0, semo1):
    wid = lax.axis_index("s") * 2 + lax.axis_index("c")
    b = wid // _NE
    e = wid % _NE
    c0 = e * _CE

    def issue_in(rb, ib, tb, sem):
        pltpu.async_copy(
            in_hbm.at[b, pl.ds(c0, _CE), pl.ds(rb * _RB, _RB), :], ib, sem)
        pltpu.async_copy(
            t_hbm.at[b, pl.ds(c0, _CE), pl.ds(rb * _RB, _RB), :], tb, sem)

    def drain_in(rb, ib, tb, sem):
        pltpu.make_async_copy(
            in_hbm.at[b, pl.ds(c0, _CE), pl.ds(rb * _RB, _RB), :], ib,
            sem).wait()
        pltpu.make_async_copy(
            t_hbm.at[b, pl.ds(c0, _CE), pl.ds(rb * _RB, _RB), :], tb,
            sem).wait()

    def issue_out(rb, s_si, s_st, s_ac, sem):
        pltpu.async_copy(s_si, si_hbm.at[e, b, pl.ds(rb * _RB, _RB), :], sem)
        pltpu.async_copy(s_st, st_hbm.at[e, b, pl.ds(rb * _RB, _RB), :], sem)
        pltpu.async_copy(s_ac, ac_hbm.at[e, b, pl.ds(rb * _RB, _RB), :], sem)

    def drain_out(s_si, s_st, s_ac, sem):
        pltpu.make_async_copy(s_si, si_hbm.at[e, b, pl.ds(0, _RB), :],
                              sem).wait()
        pltpu.make_async_copy(s_st, st_hbm.at[e, b, pl.ds(0, _RB), :],
                              sem).wait()
        pltpu.make_async_copy(s_ac, ac_hbm.at[e, b, pl.ds(0, _RB), :],
                              sem).wait()

    zero = jnp.zeros((16,), jnp.float32)

    def compute(ib, tb, s_si, s_st, s_ac):
        def row_body(r, carry):
            @plsc.parallel_loop(0, _NG, step=1, unroll=2,
                                carry=jnp.int32(0))
            def grp_body(o, dummy):
                off = o * 16
                si = zero
                st = zero
                ac = zero
                for c in range(_CE):
                    iv = ib[c, r, pl.ds(off, 16)]
                    tv = tb[c, r, pl.ds(off, 16)]
                    te = jnp.exp(tv)
                    si = si + jnp.exp(iv)
                    st = st + te
                    ac = ac + te * (tv - iv)
                s_si[r, pl.ds(off, 16)] = si
                s_st[r, pl.ds(off, 16)] = st
                s_ac[r, pl.ds(off, 16)] = ac
                return dummy

            return carry

        lax.fori_loop(0, _RB, row_body, jnp.int32(0))

    issue_in(0, i0, t0, semi0)
    drain_in(0, i0, t0, semi0)
    issue_in(1, i1, t1, semi1)
    compute(i0, t0, a_si, a_st, a_ac)
    issue_out(0, a_si, a_st, a_ac, semo0)
    drain_in(1, i1, t1, semi1)
    issue_in(2, i0, t0, semi0)
    compute(i1, t1, b_si, b_st, b_ac)
    issue_out(1, b_si, b_st, b_ac, semo1)

    def pair_body(k, carry):
        rb_a = 2 * k
        rb_b = rb_a + 1
        rb_n = jnp.minimum(rb_a + 2, _NRB - 1)
        drain_in(rb_a, i0, t0, semi0)
        issue_in(rb_b, i1, t1, semi1)
        drain_out(a_si, a_st, a_ac, semo0)
        compute(i0, t0, a_si, a_st, a_ac)
        issue_out(rb_a, a_si, a_st, a_ac, semo0)
        drain_in(rb_b, i1, t1, semi1)
        issue_in(rb_n, i0, t0, semi0)
        drain_out(b_si, b_st, b_ac, semo1)
        compute(i1, t1, b_si, b_st, b_ac)
        issue_out(rb_b, b_si, b_st, b_ac, semo1)
        return carry

    lax.fori_loop(1, _NRB // 2, pair_body, jnp.int32(0))
    drain_in(_NRB - 1, i0, t0, semi0)
    drain_out(a_si, a_st, a_ac, semo0)
    drain_out(b_si, b_st, b_ac, semo1)


def _tc_body(in_ref, t_ref, lab_ref, s_ref, n_ref):
    i = pl.program_id(0)
    j = pl.program_id(1)

    @pl.when((i == 0) & (j == 0))
    def _():
        s_ref[...] = jnp.zeros_like(s_ref)
        n_ref[...] = jnp.zeros_like(n_ref)

    iv = in_ref[0]
    tv = t_ref[0]
    te = jnp.exp(tv)
    si = jnp.sum(jnp.exp(iv), axis=0)
    st = jnp.sum(te, axis=0)
    ac = jnp.sum(te * (tv - iv), axis=0)
    kl = ac / st + jnp.log(si) - jnp.log(st)
    m = lab_ref[0] != 0
    s_ref[...] += jnp.sum(jnp.where(m, kl, 0.0))[None, None]
    n_ref[...] += jnp.sum(jnp.where(m, 1.0, 0.0))[None, None]


def _p2_body(si_ref, st_ref, ac_ref, lab_ref, ts_ref, tn_ref,
             s_ref, n_ref, l_ref):
    i = pl.program_id(0)
    j = pl.program_id(1)

    @pl.when((i == 0) & (j == 0))
    def _():
        s_ref[...] = jnp.zeros_like(s_ref)
        n_ref[...] = jnp.zeros_like(n_ref)

    si = jnp.sum(si_ref[...], axis=0)
    st = jnp.sum(st_ref[...], axis=0)
    ac = jnp.sum(ac_ref[...], axis=0)
    kl = ac / st + jnp.log(si) - jnp.log(st)
    m = lab_ref[...] != 0
    s_ref[...] += jnp.sum(jnp.where(m, kl, 0.0))[None, None]
    n_ref[...] += jnp.sum(jnp.where(m, 1.0, 0.0))[None, None]

    @pl.when((i == _B - 1) & (j == pl.num_programs(1) - 1))
    def _():
        l_ref[...] = ((s_ref[...] + ts_ref[...])
                      / (n_ref[...] + tn_ref[...]))


def kernel(input, target, label):
    lab = label.astype(jnp.int32)

    mesh = plsc.VectorSubcoreMesh(core_axis_name="c", subcore_axis_name="s")
    p1 = functools.partial(
        pl.kernel,
        mesh=mesh,
        out_type=[
            jax.ShapeDtypeStruct((_NE, _B, _RS, _W), jnp.float32),
            jax.ShapeDtypeStruct((_NE, _B, _RS, _W), jnp.float32),
            jax.ShapeDtypeStruct((_NE, _B, _RS, _W), jnp.float32),
        ],
        scratch_types=[
            pltpu.VMEM((_CE, _RB, _W), jnp.float32),
            pltpu.VMEM((_CE, _RB, _W), jnp.float32),
            pltpu.VMEM((_CE, _RB, _W), jnp.float32),
            pltpu.VMEM((_CE, _RB, _W), jnp.float32),
            pltpu.VMEM((_RB, _W), jnp.float32),
            pltpu.VMEM((_RB, _W), jnp.float32),
            pltpu.VMEM((_RB, _W), jnp.float32),
            pltpu.VMEM((_RB, _W), jnp.float32),
            pltpu.VMEM((_RB, _W), jnp.float32),
            pltpu.VMEM((_RB, _W), jnp.float32),
            pltpu.SemaphoreType.DMA,
            pltpu.SemaphoreType.DMA,
            pltpu.SemaphoreType.DMA,
            pltpu.SemaphoreType.DMA,
        ],
    )(_p1_body)
    si_p, st_p, ac_p = p1(input, target)

    nb = (_H - _RS) // _TCH
    ts2d, tn2d = pl.pallas_call(
        _tc_body,
        grid=(_B, nb),
        in_specs=[
            pl.BlockSpec((1, _C, _TCH, _W),
                         lambda i, j: (i, 0, _RS // _TCH + j, 0)),
            pl.BlockSpec((1, _C, _TCH, _W),
                         lambda i, j: (i, 0, _RS // _TCH + j, 0)),
            pl.BlockSpec((1, _TCH, _W), lambda i, j: (i, _RS // _TCH + j, 0)),
        ],
        out_specs=[
            pl.BlockSpec((1, 1), lambda i, j: (0, 0)),
            pl.BlockSpec((1, 1), lambda i, j: (0, 0)),
        ],
        out_shape=[
            jax.ShapeDtypeStruct((1, 1), jnp.float32),
            jax.ShapeDtypeStruct((1, 1), jnp.float32),
        ],
    )(input, target, lab)

    rows = 56
    grid = (_B, _RS // rows)
    _, _, loss2d = pl.pallas_call(
        _p2_body,
        grid=grid,
        in_specs=[
            pl.BlockSpec((_NE, 1, rows, _W), lambda i, j: (0, i, j, 0)),
            pl.BlockSpec((_NE, 1, rows, _W), lambda i, j: (0, i, j, 0)),
            pl.BlockSpec((_NE, 1, rows, _W), lambda i, j: (0, i, j, 0)),
            pl.BlockSpec((1, rows, _W), lambda i, j: (i, j, 0)),
            pl.BlockSpec((1, 1), lambda i, j: (0, 0)),
            pl.BlockSpec((1, 1), lambda i, j: (0, 0)),
        ],
        out_specs=[
            pl.BlockSpec((1, 1), lambda i, j: (0, 0)),
            pl.BlockSpec((1, 1), lambda i, j: (0, 0)),
            pl.BlockSpec((1, 1), lambda i, j: (0, 0)),
        ],
        out_shape=[
            jax.ShapeDtypeStruct((1, 1), jnp.float32),
            jax.ShapeDtypeStruct((1, 1), jnp.float32),
            jax.ShapeDtypeStruct((1, 1), jnp.float32),
        ],
    )(si_p, st_p, ac_p, lab, ts2d, tn2d)
    return loss2d[0, 0]
